# baseline (speedup 1.0000x reference)
import functools

import jax
import jax.numpy as jnp
from jax.experimental import pallas as pl
from jax.experimental.pallas import tpu as pltpu

_D = 1024
_E = 16
_K = 8
_F = 2 * _D
_FC = 2048
_NF = _F // _FC
_S = 4


def _gate_body(x_ref, w1_ref, b1_ref, w2_ref, b2_ref,
               probs_ref, idx_ref, gates_ref):
    x = x_ref[...]
    h = jnp.maximum(
        jnp.dot(x, w1_ref[...], preferred_element_type=jnp.float32)
        + b1_ref[...], 0.0)
    s = jnp.dot(h, w2_ref[...], preferred_element_type=jnp.float32) \
        + b2_ref[...]
    m = jnp.max(s, axis=1, keepdims=True)
    e = jnp.exp(s - m)
    probs = e / jnp.sum(e, axis=1, keepdims=True)
    probs_ref[...] = probs

    iota_e = jax.lax.broadcasted_iota(jnp.int32, (1, _E), 1)
    iota_k = jax.lax.broadcasted_iota(jnp.int32, (1, _K), 1)
    p = probs
    vals = jnp.zeros((1, _K), jnp.float32)
    idxs = jnp.zeros((1, _K), jnp.int32)
    for i in range(_K):
        mv = jnp.max(p, axis=1, keepdims=True)
        ai = jnp.min(jnp.where(p == mv, iota_e, _E), axis=1, keepdims=True)
        vals = jnp.where(iota_k == i, mv, vals)
        idxs = jnp.where(iota_k == i, ai, idxs)
        p = jnp.where(iota_e == ai, -jnp.inf, p)
    idx_ref[...] = idxs
    vm = jnp.max(vals, axis=1, keepdims=True)
    ev = jnp.exp(vals - vm)
    gates_ref[...] = ev / jnp.sum(ev, axis=1, keepdims=True)


def _expert_body(idx_ref, gate_ref, x_ref, *refs):
    w1_refs = refs[:_S]
    b1_ref = refs[_S]
    w2_refs = refs[_S + 1:2 * _S + 1]
    b2_ref, lw_ref, lb_ref = refs[2 * _S + 1:2 * _S + 4]
    out_ref = refs[-2]
    acc_ref = refs[-1]
    k = pl.program_id(0)
    f = pl.program_id(1)
    x = x_ref[...]
    b1 = b1_ref[0]
    h = _FC // _S
    part = None
    for s in range(_S):
        hh = jnp.dot(x, w1_refs[s][0], preferred_element_type=jnp.float32) \
            + b1[:, s * h:(s + 1) * h]
        hh = 0.5 * hh * (1.0 + jax.lax.erf(hh * 0.7071067811865476))
        p = jnp.dot(hh, w2_refs[s][0], preferred_element_type=jnp.float32)
        part = p if part is None else part + p

    @pl.when(f == 0)
    def _():
        acc_ref[...] = part + b2_ref[0]

    @pl.when(f != 0)
    def _():
        acc_ref[...] = acc_ref[...] + part

    @pl.when(f == _NF - 1)
    def _():
        oo = acc_ref[...]
        mu = jnp.mean(oo, axis=1, keepdims=True)
        d = oo - mu
        var = jnp.mean(d * d, axis=1, keepdims=True)
        nn = d * jax.lax.rsqrt(var + 1e-5) * lw_ref[0] + lb_ref[0]
        g = gate_ref[k]
        gated = g * nn

        @pl.when(k == 0)
        def _():
            out_ref[...] = gated

        @pl.when(k != 0)
        def _():
            out_ref[...] = out_ref[...] + gated


@jax.jit
def kernel(features, gate_W1, gate_b1, gate_W2, gate_b2,
           We1, be1, We2, be2, ln_w, ln_b):
    x = features.reshape(-1)[:_D].reshape(1, _D)

    probs, idxs, gates = pl.pallas_call(
        _gate_body,
        out_shape=(
            jax.ShapeDtypeStruct((1, _E), jnp.float32),
            jax.ShapeDtypeStruct((1, _K), jnp.int32),
            jax.ShapeDtypeStruct((1, _K), jnp.float32),
        ),
    )(x, gate_W1, gate_b1.reshape(1, -1), gate_W2, gate_b2.reshape(1, -1))

    grid = (_K, _NF)

    def _w1_spec(s):
        return pl.BlockSpec((1, _D, _FC // _S),
                            lambda k, f, idx, g: (idx[k], 0, _S * f + s))

    def _w2_spec(s):
        return pl.BlockSpec((1, _FC // _S, _D),
                            lambda k, f, idx, g: (idx[k], _S * f + s, 0))

    expert = pl.pallas_call(
        _expert_body,
        grid_spec=pltpu.PrefetchScalarGridSpec(
            num_scalar_prefetch=2,
            grid=grid,
            in_specs=(
                [pl.BlockSpec((1, _D), lambda k, f, idx, g: (0, 0))]
                + [_w1_spec(s) for s in range(_S)]
                + [pl.BlockSpec((1, 1, _FC),
                                lambda k, f, idx, g: (idx[k], 0, f))]
                + [_w2_spec(s) for s in range(_S)]
                + [pl.BlockSpec((1, 1, _D),
                                lambda k, f, idx, g: (idx[k], 0, 0))] * 3
            ),
            out_specs=pl.BlockSpec((1, _D), lambda k, f, idx, g: (0, 0)),
            scratch_shapes=[pltpu.VMEM((1, _D), jnp.float32)],
        ),
        out_shape=jax.ShapeDtypeStruct((1, _D), jnp.float32),
        compiler_params=pltpu.CompilerParams(
            dimension_semantics=("arbitrary", "arbitrary")),
    )(idxs.reshape(_K), gates.reshape(_K), x,
      *([We1] * _S), be1.reshape(_E, 1, _F), *([We2] * _S),
      be2.reshape(_E, 1, _D), ln_w.reshape(_E, 1, _D),
      ln_b.reshape(_E, 1, _D))

    return expert.reshape(_D), probs.reshape(_E)


# speedup vs baseline: 1.0111x; 1.0111x over previous
import jax
import jax.numpy as jnp
from jax.experimental import pallas as pl
from jax.experimental.pallas import tpu as pltpu

_D = 1024
_E = 16
_K = 8
_F = 2 * _D
_FC = 2048
_NF = _F // _FC
_S = 2


def _gate_body(x_ref, w1_ref, b1_ref, w2_ref, b2_ref,
               probs_ref, idx_ref, gates_ref):
    x = x_ref[...]
    h = jnp.maximum(
        jnp.dot(x, w1_ref[...], preferred_element_type=jnp.float32)
        + b1_ref[...], 0.0)
    s = jnp.dot(h, w2_ref[...], preferred_element_type=jnp.float32) \
        + b2_ref[...]
    m = jnp.max(s, axis=1, keepdims=True)
    e = jnp.exp(s - m)
    probs = e / jnp.sum(e, axis=1, keepdims=True)
    probs_ref[...] = probs

    iota_e = jax.lax.broadcasted_iota(jnp.int32, (1, _E), 1)
    iota_k = jax.lax.broadcasted_iota(jnp.int32, (1, _K), 1)
    p = probs
    vals = jnp.zeros((1, _K), jnp.float32)
    idxs = jnp.zeros((1, _K), jnp.int32)
    for i in range(_K):
        mv = jnp.max(p, axis=1, keepdims=True)
        ai = jnp.min(jnp.where(p == mv, iota_e, _E), axis=1, keepdims=True)
        vals = jnp.where(iota_k == i, mv, vals)
        idxs = jnp.where(iota_k == i, ai, idxs)
        p = jnp.where(iota_e == ai, -jnp.inf, p)
    idx_ref[...] = idxs
    vm = jnp.max(vals, axis=1, keepdims=True)
    ev = jnp.exp(vals - vm)
    gates_ref[...] = ev / jnp.sum(ev, axis=1, keepdims=True)


def _expert_body(idx_ref, gate_ref, x_ref, *refs):
    w1_refs = refs[:_S]
    b1_ref = refs[_S]
    w2_refs = refs[_S + 1:2 * _S + 1]
    b2_ref, lw_ref, lb_ref = refs[2 * _S + 1:2 * _S + 4]
    out_ref = refs[-2]
    acc_ref = refs[-1]
    k = pl.program_id(0)
    f = pl.program_id(1)
    x = x_ref[...]
    b1 = b1_ref[0]
    h = _FC // _S
    part = None
    for s in range(_S):
        hh = jnp.dot(x, w1_refs[s][0], preferred_element_type=jnp.float32) \
            + b1[:, s * h:(s + 1) * h]
        hh = 0.5 * hh * (1.0 + jax.lax.erf(hh * 0.7071067811865476))
        p = jnp.dot(hh, w2_refs[s][0], preferred_element_type=jnp.float32)
        part = p if part is None else part + p

    @pl.when(f == 0)
    def _():
        acc_ref[...] = part + b2_ref[0]

    @pl.when(f != 0)
    def _():
        acc_ref[...] = acc_ref[...] + part

    @pl.when(f == _NF - 1)
    def _():
        oo = acc_ref[...]
        mu = jnp.mean(oo, axis=1, keepdims=True)
        d = oo - mu
        var = jnp.mean(d * d, axis=1, keepdims=True)
        nn = d * jax.lax.rsqrt(var + 1e-5) * lw_ref[0] + lb_ref[0]
        g = gate_ref[k]
        gated = g * nn

        @pl.when(k == 0)
        def _():
            out_ref[...] = gated

        @pl.when(k != 0)
        def _():
            out_ref[...] = out_ref[...] + gated


@jax.jit
def kernel(features, gate_W1, gate_b1, gate_W2, gate_b2,
           We1, be1, We2, be2, ln_w, ln_b):
    x = features.reshape(-1)[:_D].reshape(1, _D)

    probs, idxs, gates = pl.pallas_call(
        _gate_body,
        out_shape=(
            jax.ShapeDtypeStruct((1, _E), jnp.float32),
            jax.ShapeDtypeStruct((1, _K), jnp.int32),
            jax.ShapeDtypeStruct((1, _K), jnp.float32),
        ),
    )(x, gate_W1, gate_b1.reshape(1, -1), gate_W2, gate_b2.reshape(1, -1))

    grid = (_K, _NF)

    def _w1_spec(s):
        return pl.BlockSpec((1, _D, _FC // _S),
                            lambda k, f, idx, g: (idx[k], 0, _S * f + s))

    def _w2_spec(s):
        return pl.BlockSpec((1, _FC // _S, _D),
                            lambda k, f, idx, g: (idx[k], _S * f + s, 0))

    expert = pl.pallas_call(
        _expert_body,
        grid_spec=pltpu.PrefetchScalarGridSpec(
            num_scalar_prefetch=2,
            grid=grid,
            in_specs=(
                [pl.BlockSpec((1, _D), lambda k, f, idx, g: (0, 0))]
                + [_w1_spec(s) for s in range(_S)]
                + [pl.BlockSpec((1, 1, _FC),
                                lambda k, f, idx, g: (idx[k], 0, f))]
                + [_w2_spec(s) for s in range(_S)]
                + [pl.BlockSpec((1, 1, _D),
                                lambda k, f, idx, g: (idx[k], 0, 0))] * 3
            ),
            out_specs=pl.BlockSpec((1, _D), lambda k, f, idx, g: (0, 0)),
            scratch_shapes=[pltpu.VMEM((1, _D), jnp.float32)],
        ),
        out_shape=jax.ShapeDtypeStruct((1, _D), jnp.float32),
        compiler_params=pltpu.CompilerParams(
            dimension_semantics=("arbitrary", "arbitrary")),
    )(idxs.reshape(_K), gates.reshape(_K), x,
      *([We1] * _S), be1.reshape(_E, 1, _F), *([We2] * _S),
      be2.reshape(_E, 1, _D), ln_w.reshape(_E, 1, _D),
      ln_b.reshape(_E, 1, _D))

    return expert.reshape(_D), probs.reshape(_E)


# speedup vs baseline: 1.0165x; 1.0053x over previous
import jax
import jax.numpy as jnp
from jax.experimental import pallas as pl
from jax.experimental.pallas import tpu as pltpu

_D = 1024
_E = 16
_K = 8
_F = 2 * _D
_S = 2


def _gate_body(x_ref, w1_ref, b1_ref, w2_ref, b2_ref,
               probs_ref, idx_ref, gates_ref):
    x = x_ref[...]
    h = jnp.maximum(
        jnp.dot(x, w1_ref[...], preferred_element_type=jnp.float32)
        + b1_ref[...], 0.0)
    s = jnp.dot(h, w2_ref[...], preferred_element_type=jnp.float32) \
        + b2_ref[...]
    m = jnp.max(s, axis=1, keepdims=True)
    e = jnp.exp(s - m)
    probs = e / jnp.sum(e, axis=1, keepdims=True)
    probs_ref[...] = probs

    iota_e = jax.lax.broadcasted_iota(jnp.int32, (1, _E), 1)
    iota_k = jax.lax.broadcasted_iota(jnp.int32, (1, _K), 1)
    p = probs
    vals = jnp.zeros((1, _K), jnp.float32)
    idxs = jnp.zeros((1, _K), jnp.int32)
    for i in range(_K):
        mv = jnp.max(p, axis=1, keepdims=True)
        ai = jnp.min(jnp.where(p == mv, iota_e, _E), axis=1, keepdims=True)
        vals = jnp.where(iota_k == i, mv, vals)
        idxs = jnp.where(iota_k == i, ai, idxs)
        p = jnp.where(iota_e == ai, -jnp.inf, p)
    idx_ref[...] = idxs
    vm = jnp.max(vals, axis=1, keepdims=True)
    ev = jnp.exp(vals - vm)
    gates_ref[...] = ev / jnp.sum(ev, axis=1, keepdims=True)


def _expert_body(idx_ref, gate_ref, x_ref, *refs):
    w1_refs = refs[:_S]
    b1_ref = refs[_S]
    w2_refs = refs[_S + 1:2 * _S + 1]
    b2_ref, lw_ref, lb_ref = refs[2 * _S + 1:2 * _S + 4]
    out_ref = refs[-1]
    k = pl.program_id(0)
    x = x_ref[...]
    b1 = b1_ref[0]
    h = _F // _S
    part = None
    for s in range(_S):
        hh = jnp.dot(x, w1_refs[s][0], preferred_element_type=jnp.float32) \
            + b1[:, s * h:(s + 1) * h]
        hh = 0.5 * hh * (1.0 + jax.lax.erf(hh * 0.7071067811865476))
        p = jnp.dot(hh, w2_refs[s][0], preferred_element_type=jnp.float32)
        part = p if part is None else part + p

    oo = part + b2_ref[0]
    mu = jnp.mean(oo, axis=1, keepdims=True)
    d = oo - mu
    var = jnp.mean(d * d, axis=1, keepdims=True)
    nn = d * jax.lax.rsqrt(var + 1e-5) * lw_ref[0] + lb_ref[0]
    gated = gate_ref[k] * nn

    @pl.when(k == 0)
    def _():
        out_ref[...] = gated

    @pl.when(k != 0)
    def _():
        out_ref[...] = out_ref[...] + gated


@jax.jit
def kernel(features, gate_W1, gate_b1, gate_W2, gate_b2,
           We1, be1, We2, be2, ln_w, ln_b):
    x = features.reshape(-1)[:_D].reshape(1, _D)

    probs, idxs, gates = pl.pallas_call(
        _gate_body,
        out_shape=(
            jax.ShapeDtypeStruct((1, _E), jnp.float32),
            jax.ShapeDtypeStruct((1, _K), jnp.int32),
            jax.ShapeDtypeStruct((1, _K), jnp.float32),
        ),
    )(x, gate_W1, gate_b1.reshape(1, -1), gate_W2, gate_b2.reshape(1, -1))

    def _w1_spec(s):
        return pl.BlockSpec((1, _D, _F // _S),
                            lambda k, idx, g: (idx[k], 0, s))

    def _w2_spec(s):
        return pl.BlockSpec((1, _F // _S, _D),
                            lambda k, idx, g: (idx[k], s, 0))

    expert = pl.pallas_call(
        _expert_body,
        grid_spec=pltpu.PrefetchScalarGridSpec(
            num_scalar_prefetch=2,
            grid=(_K,),
            in_specs=(
                [pl.BlockSpec((1, _D), lambda k, idx, g: (0, 0))]
                + [_w1_spec(s) for s in range(_S)]
                + [pl.BlockSpec((1, 1, _F), lambda k, idx, g: (idx[k], 0, 0))]
                + [_w2_spec(s) for s in range(_S)]
                + [pl.BlockSpec((1, 1, _D),
                                lambda k, idx, g: (idx[k], 0, 0))] * 3
            ),
            out_specs=pl.BlockSpec((1, _D), lambda k, idx, g: (0, 0)),
        ),
        out_shape=jax.ShapeDtypeStruct((1, _D), jnp.float32),
        compiler_params=pltpu.CompilerParams(
            dimension_semantics=("arbitrary",)),
    )(idxs.reshape(_K), gates.reshape(_K), x,
      *([We1] * _S), be1.reshape(_E, 1, _F), *([We2] * _S),
      be2.reshape(_E, 1, _D), ln_w.reshape(_E, 1, _D),
      ln_b.reshape(_E, 1, _D))

    return expert.reshape(_D), probs.reshape(_E)


# speedup vs baseline: 1.0458x; 1.0289x over previous
import jax
import jax.numpy as jnp
from jax.experimental import pallas as pl
from jax.experimental.pallas import tpu as pltpu

_D = 1024
_E = 16
_K = 8
_F = 2 * _D


def _fused_body(x_ref, gw1_ref, gb1_ref, gw2_ref, gb2_ref,
                we1_hbm, be1_hbm, we2_hbm, be2_hbm, lw_hbm, lb_hbm,
                probs_ref, out_ref,
                w1buf, w2buf, b1buf, b2buf, lwbuf, lbbuf, sems):
    x = x_ref[...]
    h = jnp.maximum(
        jnp.dot(x, gw1_ref[...], preferred_element_type=jnp.float32)
        + gb1_ref[...], 0.0)
    s = jnp.dot(h, gw2_ref[...], preferred_element_type=jnp.float32) \
        + gb2_ref[...]
    m = jnp.max(s, axis=1, keepdims=True)
    e = jnp.exp(s - m)
    probs = e / jnp.sum(e, axis=1, keepdims=True)
    probs_ref[...] = probs

    iota_e = jax.lax.broadcasted_iota(jnp.int32, (1, _E), 1)
    iota_k = jax.lax.broadcasted_iota(jnp.int32, (1, _K), 1)
    p = probs
    vals = jnp.zeros((1, _K), jnp.float32)
    ais = []
    for i in range(_K):
        mv = jnp.max(p, axis=1, keepdims=True)
        ai = jnp.min(jnp.where(p == mv, iota_e, _E))
        vals = jnp.where(iota_k == i, mv, vals)
        ais.append(ai)
        p = jnp.where(iota_e == ai, -jnp.inf, p)
    vm = jnp.max(vals, axis=1, keepdims=True)
    ev = jnp.exp(vals - vm)
    gates = ev / jnp.sum(ev, axis=1, keepdims=True)

    def copies(slot, eidx):
        return [
            pltpu.make_async_copy(we1_hbm.at[eidx], w1buf.at[slot],
                                  sems.at[0, slot]),
            pltpu.make_async_copy(we2_hbm.at[eidx], w2buf.at[slot],
                                  sems.at[1, slot]),
            pltpu.make_async_copy(be1_hbm.at[eidx], b1buf.at[slot],
                                  sems.at[2, slot]),
            pltpu.make_async_copy(be2_hbm.at[eidx], b2buf.at[slot],
                                  sems.at[3, slot]),
            pltpu.make_async_copy(lw_hbm.at[eidx], lwbuf.at[slot],
                                  sems.at[4, slot]),
            pltpu.make_async_copy(lb_hbm.at[eidx], lbbuf.at[slot],
                                  sems.at[5, slot]),
        ]

    for c in copies(0, ais[0]):
        c.start()

    acc = jnp.zeros((1, _D), jnp.float32)
    for k in range(_K):
        slot = k % 2
        if k + 1 < _K:
            for c in copies((k + 1) % 2, ais[k + 1]):
                c.start()
        for c in copies(slot, ais[k]):
            c.wait()
        w1 = w1buf[slot]
        w2 = w2buf[slot]
        hh = jnp.dot(x, w1, preferred_element_type=jnp.float32) + b1buf[slot]
        hh = 0.5 * hh * (1.0 + jax.lax.erf(hh * 0.7071067811865476))
        oo = jnp.dot(hh, w2, preferred_element_type=jnp.float32) + b2buf[slot]
        mu = jnp.mean(oo, axis=1, keepdims=True)
        d = oo - mu
        var = jnp.mean(d * d, axis=1, keepdims=True)
        nn = d * jax.lax.rsqrt(var + 1e-5) * lwbuf[slot] + lbbuf[slot]
        acc = acc + gates[:, k:k + 1] * nn
    out_ref[...] = acc


@jax.jit
def kernel(features, gate_W1, gate_b1, gate_W2, gate_b2,
           We1, be1, We2, be2, ln_w, ln_b):
    x = features.reshape(-1)[:_D].reshape(1, _D)
    hbm = pl.BlockSpec(memory_space=pltpu.MemorySpace.HBM)
    probs, out = pl.pallas_call(
        _fused_body,
        in_specs=[
            pl.BlockSpec((1, _D), lambda: (0, 0)),
            pl.BlockSpec((_D, _D // 2), lambda: (0, 0)),
            pl.BlockSpec((1, _D // 2), lambda: (0, 0)),
            pl.BlockSpec((_D // 2, _E), lambda: (0, 0)),
            pl.BlockSpec((1, _E), lambda: (0, 0)),
            hbm, hbm, hbm, hbm, hbm, hbm,
        ],
        out_specs=(
            pl.BlockSpec((1, _E), lambda: (0, 0)),
            pl.BlockSpec((1, _D), lambda: (0, 0)),
        ),
        out_shape=(
            jax.ShapeDtypeStruct((1, _E), jnp.float32),
            jax.ShapeDtypeStruct((1, _D), jnp.float32),
        ),
        scratch_shapes=[
            pltpu.VMEM((2, _D, _F), jnp.float32),
            pltpu.VMEM((2, _F, _D), jnp.float32),
            pltpu.VMEM((2, 1, _F), jnp.float32),
            pltpu.VMEM((2, 1, _D), jnp.float32),
            pltpu.VMEM((2, 1, _D), jnp.float32),
            pltpu.VMEM((2, 1, _D), jnp.float32),
            pltpu.SemaphoreType.DMA((6, 2)),
        ],
    )(x, gate_W1, gate_b1.reshape(1, -1), gate_W2, gate_b2.reshape(1, -1),
      We1, be1.reshape(_E, 1, _F), We2, be2.reshape(_E, 1, _D),
      ln_w.reshape(_E, 1, _D), ln_b.reshape(_E, 1, _D))
    return out.reshape(_D), probs.reshape(_E)
